# initial kernel scaffold (unmeasured)
import jax
import jax.numpy as jnp
from jax import lax
from jax.experimental import pallas as pl
from jax.experimental.pallas import tpu as pltpu


def kernel(
    x,
):
    def body(*refs):
        pass

    out_shape = jax.ShapeDtypeStruct(..., jnp.float32)
    return pl.pallas_call(body, out_shape=out_shape)(...)


# baseline (device time: 895485 ns/iter reference)
import jax
import jax.numpy as jnp
from jax import lax
from jax.experimental import pallas as pl
from jax.experimental.pallas import tpu as pltpu

C = 2048


def kernel(x):
    m_per, n = x.shape
    assert m_per % C == 0
    nchunk = m_per // C

    def body(x_hbm, out_hbm, local_f32, send_bf, recv_bf, res_bf,
             in_sems, out_sems, send_sems, recv_sems, credit_sem):
        mx = lax.axis_index("x")
        my = lax.axis_index("y")
        mz = lax.axis_index("z")
        partner = (mx, my, 1 - mz)

        barrier = pltpu.get_barrier_semaphore()
        pl.semaphore_signal(barrier, inc=1, device_id=partner,
                            device_id_type=pl.DeviceIdType.MESH)
        pl.semaphore_wait(barrier, 1)

        for j in range(nchunk):
            slot = j % 2
            rows = pl.ds(j * C, C)

            load = pltpu.make_async_copy(
                x_hbm.at[rows, :], local_f32.at[slot], in_sems.at[slot])
            load.start()
            load.wait()
            send_bf[slot] = local_f32[slot].astype(jnp.bfloat16)

            rdma = pltpu.make_async_remote_copy(
                src_ref=send_bf.at[slot],
                dst_ref=recv_bf.at[slot],
                send_sem=send_sems.at[slot],
                recv_sem=recv_sems.at[slot],
                device_id=partner,
                device_id_type=pl.DeviceIdType.MESH,
            )
            if j >= 2:
                pl.semaphore_wait(credit_sem, 1)
            rdma.start()
            rdma.wait()

            res_bf[slot] = (local_f32[slot]
                            + recv_bf[slot].astype(jnp.float32)
                            ).astype(jnp.bfloat16)
            if j + 2 < nchunk:
                pl.semaphore_signal(credit_sem, inc=1, device_id=partner,
                                    device_id_type=pl.DeviceIdType.MESH)

            store = pltpu.make_async_copy(
                res_bf.at[slot], out_hbm.at[rows, :], out_sems.at[slot])
            store.start()
            store.wait()

    return pl.pallas_call(
        body,
        out_shape=jax.ShapeDtypeStruct((m_per, n), jnp.bfloat16),
        in_specs=[pl.BlockSpec(memory_space=pl.ANY)],
        out_specs=pl.BlockSpec(memory_space=pl.ANY),
        scratch_shapes=[
            pltpu.VMEM((2, C, n), jnp.float32),
            pltpu.VMEM((2, C, n), jnp.bfloat16),
            pltpu.VMEM((2, C, n), jnp.bfloat16),
            pltpu.VMEM((2, C, n), jnp.bfloat16),
            pltpu.SemaphoreType.DMA((2,)),
            pltpu.SemaphoreType.DMA((2,)),
            pltpu.SemaphoreType.DMA((2,)),
            pltpu.SemaphoreType.DMA((2,)),
            pltpu.SemaphoreType.REGULAR,
        ],
        compiler_params=pltpu.CompilerParams(
            collective_id=0, vmem_limit_bytes=100 * 1024 * 1024),
    )(x)


# device time: 771886 ns/iter; 1.1601x vs baseline; 1.1601x over previous
import jax
import jax.numpy as jnp
from jax import lax
from jax.experimental import pallas as pl
from jax.experimental.pallas import tpu as pltpu

C = 2048


def kernel(x):
    m_per, n = x.shape
    assert m_per % C == 0
    nchunk = m_per // C
    assert nchunk >= 4

    def body(x_hbm, out_hbm, local_f32, send_bf, recv_bf, res_bf,
             in_sems, out_sems, send_sems, recv_sems, credit_sem):
        mx = lax.axis_index("x")
        my = lax.axis_index("y")
        mz = lax.axis_index("z")
        partner = (mx, my, 1 - mz)

        def make_rdma(j):
            return pltpu.make_async_remote_copy(
                src_ref=send_bf.at[j % 2],
                dst_ref=recv_bf.at[j % 2],
                send_sem=send_sems.at[j % 2],
                recv_sem=recv_sems.at[j % 2],
                device_id=partner,
                device_id_type=pl.DeviceIdType.MESH,
            )

        def add_store(j):
            res_bf[j % 2] = (local_f32[j % 2]
                             + recv_bf[j % 2].astype(jnp.float32)
                             ).astype(jnp.bfloat16)
            cp = pltpu.make_async_copy(
                res_bf.at[j % 2], out_hbm.at[pl.ds(j * C, C), :],
                out_sems.at[j % 2])
            cp.start()
            cp.wait()

        barrier = pltpu.get_barrier_semaphore()
        pl.semaphore_signal(barrier, inc=1, device_id=partner,
                            device_id_type=pl.DeviceIdType.MESH)
        pl.semaphore_wait(barrier, 1)

        rdmas = {}
        for j in range(nchunk):
            slot = j % 2
            load = pltpu.make_async_copy(
                x_hbm.at[pl.ds(j * C, C), :], local_f32.at[slot],
                in_sems.at[slot])
            load.start()
            load.wait()
            if j >= 2:
                rdmas[j - 2].wait_send()
            send_bf[slot] = local_f32[slot].astype(jnp.bfloat16)
            if j >= 2:
                pl.semaphore_wait(credit_sem, 1)
            rdmas[j] = make_rdma(j)
            rdmas[j].start()

            if j >= 1:
                rdmas[j - 1].wait_recv()
                add_store(j - 1)
                if j + 1 < nchunk:
                    pl.semaphore_signal(credit_sem, inc=1,
                                        device_id=partner,
                                        device_id_type=pl.DeviceIdType.MESH)

        last = nchunk - 1
        rdmas[last].wait_recv()
        add_store(last)
        rdmas[last - 1].wait_send()
        rdmas[last].wait_send()

    return pl.pallas_call(
        body,
        out_shape=jax.ShapeDtypeStruct((m_per, n), jnp.bfloat16),
        in_specs=[pl.BlockSpec(memory_space=pl.ANY)],
        out_specs=pl.BlockSpec(memory_space=pl.ANY),
        scratch_shapes=[
            pltpu.VMEM((2, C, n), jnp.float32),
            pltpu.VMEM((2, C, n), jnp.bfloat16),
            pltpu.VMEM((2, C, n), jnp.bfloat16),
            pltpu.VMEM((2, C, n), jnp.bfloat16),
            pltpu.SemaphoreType.DMA((2,)),
            pltpu.SemaphoreType.DMA((2,)),
            pltpu.SemaphoreType.DMA((2,)),
            pltpu.SemaphoreType.DMA((2,)),
            pltpu.SemaphoreType.REGULAR,
        ],
        compiler_params=pltpu.CompilerParams(
            collective_id=0, vmem_limit_bytes=100 * 1024 * 1024),
    )(x)


# device time: 373746 ns/iter; 2.3960x vs baseline; 2.0653x over previous
import jax
import jax.numpy as jnp
from jax import lax
from jax.experimental import pallas as pl
from jax.experimental.pallas import tpu as pltpu

C = 1024
MESH = pl.DeviceIdType.MESH


def kernel(x):
    m_per, n = x.shape
    Q = m_per // 4
    K = Q // C
    H = C // 2
    assert Q % C == 0 and K >= 3

    def body(x_hbm, out_hbm,
             local_f32, zsend, zrecv, rbuf, xrecv, yrecv, dxrecv, dyrecv,
             in_sems, out_sems,
             z_ssem, z_rsem, xr_ssem, xr_rsem, yr_ssem, yr_rsem,
             xf_ssem, xf_rsem, yf_ssem, yf_rsem,
             z_cr, xr_cr, yr_cr, xf_cr, yf_cr):
        mx = lax.axis_index("x")
        my = lax.axis_index("y")
        mz = lax.axis_index("z")
        zp = (mx, my, 1 - mz)
        xn = (1 - mx, my, mz)
        yn = (mx, 1 - my, mz)
        q_me = 2 * mx + my
        q_xn = 2 * (1 - mx) + my
        q_yn = 2 * mx + (1 - my)
        q_dg = 2 * (1 - mx) + (1 - my)

        def sig(sem, nbr):
            pl.semaphore_signal(sem, inc=1, device_id=nbr,
                                device_id_type=MESH)

        barrier = pltpu.get_barrier_semaphore()
        for nbr in (zp, xn, yn):
            sig(barrier, nbr)
        pl.semaphore_wait(barrier, 3)

        zr, xr, yr, xf, yf = {}, {}, {}, {}, {}
        st_r, st_x, st_y, st_dx, st_dy = {}, {}, {}, {}, {}

        for it in range(K + 3):
            if it < K:
                k, s = it, it % 2
                load = pltpu.make_async_copy(
                    x_hbm.at[pl.ds(q_me * Q + k * C, C), :],
                    local_f32.at[s], in_sems.at[s])
                load.start()
                load.wait()
                zsend[s] = local_f32[s].astype(jnp.bfloat16)
                if k >= 2:
                    pl.semaphore_wait(z_cr, 1)
                zr[k] = pltpu.make_async_remote_copy(
                    zsend.at[s], zrecv.at[s], z_ssem.at[s], z_rsem.at[s],
                    device_id=zp, device_id_type=MESH)
                zr[k].start()

            if 0 <= it - 1 < K:
                k, s = it - 1, (it - 1) % 2
                zr[k].wait()
                rbuf[s] = (local_f32[s] + zrecv[s].astype(jnp.float32)
                           ).astype(jnp.bfloat16)
                if k + 2 < K:
                    sig(z_cr, zp)
                st_r[k] = pltpu.make_async_copy(
                    rbuf.at[s], out_hbm.at[pl.ds(q_me * Q + k * C, C), :],
                    out_sems.at[0, s])
                st_r[k].start()
                if k >= 2:
                    pl.semaphore_wait(xr_cr, 1)
                    pl.semaphore_wait(yr_cr, 1)
                xr[k] = pltpu.make_async_remote_copy(
                    rbuf.at[s], xrecv.at[s], xr_ssem.at[s], xr_rsem.at[s],
                    device_id=xn, device_id_type=MESH)
                yr[k] = pltpu.make_async_remote_copy(
                    rbuf.at[s], yrecv.at[s], yr_ssem.at[s], yr_rsem.at[s],
                    device_id=yn, device_id_type=MESH)
                xr[k].start()
                yr[k].start()

            if 0 <= it - 2 < K:
                k, s = it - 2, (it - 2) % 2
                xr[k].wait()
                yr[k].wait()
                st_x[k] = pltpu.make_async_copy(
                    xrecv.at[s], out_hbm.at[pl.ds(q_xn * Q + k * C, C), :],
                    out_sems.at[1, s])
                st_y[k] = pltpu.make_async_copy(
                    yrecv.at[s], out_hbm.at[pl.ds(q_yn * Q + k * C, C), :],
                    out_sems.at[2, s])
                st_x[k].start()
                st_y[k].start()
                if k >= 2:
                    pl.semaphore_wait(xf_cr, 1)
                    pl.semaphore_wait(yf_cr, 1)
                xf[k] = pltpu.make_async_remote_copy(
                    yrecv.at[s, pl.ds(0, H), :], dxrecv.at[s],
                    xf_ssem.at[s], xf_rsem.at[s],
                    device_id=xn, device_id_type=MESH)
                yf[k] = pltpu.make_async_remote_copy(
                    xrecv.at[s, pl.ds(H, H), :], dyrecv.at[s],
                    yf_ssem.at[s], yf_rsem.at[s],
                    device_id=yn, device_id_type=MESH)
                xf[k].start()
                yf[k].start()
                xf[k].wait_send()
                yf[k].wait_send()
                st_x[k].wait()
                st_y[k].wait()
                st_r[k].wait()
                if k + 2 < K:
                    sig(xr_cr, xn)
                    sig(yr_cr, yn)

            if 0 <= it - 3 < K:
                k, s = it - 3, (it - 3) % 2
                xf[k].wait_recv()
                yf[k].wait_recv()
                st_dx[k] = pltpu.make_async_copy(
                    dxrecv.at[s], out_hbm.at[pl.ds(q_dg * Q + k * C, H), :],
                    out_sems.at[3, s])
                st_dy[k] = pltpu.make_async_copy(
                    dyrecv.at[s],
                    out_hbm.at[pl.ds(q_dg * Q + k * C + H, H), :],
                    out_sems.at[4, s])
                st_dx[k].start()
                st_dy[k].start()
                st_dx[k].wait()
                st_dy[k].wait()
                if k + 2 < K:
                    sig(xf_cr, xn)
                    sig(yf_cr, yn)

    return pl.pallas_call(
        body,
        out_shape=jax.ShapeDtypeStruct((m_per, n), jnp.bfloat16),
        in_specs=[pl.BlockSpec(memory_space=pl.ANY)],
        out_specs=pl.BlockSpec(memory_space=pl.ANY),
        scratch_shapes=[
            pltpu.VMEM((2, C, n), jnp.float32),
            pltpu.VMEM((2, C, n), jnp.bfloat16),
            pltpu.VMEM((2, C, n), jnp.bfloat16),
            pltpu.VMEM((2, C, n), jnp.bfloat16),
            pltpu.VMEM((2, C, n), jnp.bfloat16),
            pltpu.VMEM((2, C, n), jnp.bfloat16),
            pltpu.VMEM((2, C // 2, n), jnp.bfloat16),
            pltpu.VMEM((2, C // 2, n), jnp.bfloat16),
            pltpu.SemaphoreType.DMA((2,)),
            pltpu.SemaphoreType.DMA((5, 2)),
            pltpu.SemaphoreType.DMA((2,)),
            pltpu.SemaphoreType.DMA((2,)),
            pltpu.SemaphoreType.DMA((2,)),
            pltpu.SemaphoreType.DMA((2,)),
            pltpu.SemaphoreType.DMA((2,)),
            pltpu.SemaphoreType.DMA((2,)),
            pltpu.SemaphoreType.DMA((2,)),
            pltpu.SemaphoreType.DMA((2,)),
            pltpu.SemaphoreType.DMA((2,)),
            pltpu.SemaphoreType.DMA((2,)),
            pltpu.SemaphoreType.REGULAR,
            pltpu.SemaphoreType.REGULAR,
            pltpu.SemaphoreType.REGULAR,
            pltpu.SemaphoreType.REGULAR,
            pltpu.SemaphoreType.REGULAR,
        ],
        compiler_params=pltpu.CompilerParams(
            collective_id=0, vmem_limit_bytes=100 * 1024 * 1024),
    )(x)
